# baseline (speedup 1.0000x reference)
import functools

import jax
import jax.numpy as jnp
from jax import lax
from jax.experimental import pallas as pl
from jax.experimental.pallas import tpu as pltpu
from jax.experimental.pallas import tpu_sc as plsc

VOCAB = 100000
N_PREFIX = 200
EMBED_DIM = 64
BATCH = 4096
HIST = 200
PAD_DIM = 128
GROUP = 128
HGROUP = GROUP // 2

NC = 2
NS = 16
NW = NC * NS

NBUF = 4
LEAD = 2
VLEN = 16


def _sc_gather(table_pad, idx_groups):
    ngroups = idx_groups.shape[0]
    GPW = ngroups // NW
    NITER = GPW // NBUF
    mesh = plsc.VectorSubcoreMesh(
        core_axis_name="c", subcore_axis_name="s", num_cores=NC, num_subcores=NS
    )

    @functools.partial(
        pl.kernel,
        out_type=jax.ShapeDtypeStruct((ngroups * HGROUP, PAD_DIM), jnp.float32),
        mesh=mesh,
        compiler_params=pltpu.CompilerParams(use_tc_tiling_on_sc=True),
        scratch_types=[
            pltpu.VMEM((GPW, GROUP), jnp.int32),
            pltpu.VMEM((NBUF, GROUP, PAD_DIM), jnp.float32),
            pltpu.VMEM((NBUF, HGROUP, PAD_DIM), jnp.float32),
        ]
        + [pltpu.SemaphoreType.DMA] * (2 * NBUF),
    )
    def body(table_hbm, idx_hbm, out_hbm, idx_t, rows, packed, *sems):
        gsems = sems[:NBUF]
        ssems = sems[NBUF:]
        wid = lax.axis_index("s") * NC + lax.axis_index("c")
        gbase = wid * GPW

        pltpu.sync_copy(idx_hbm.at[pl.ds(gbase, GPW)], idx_t)

        def start_gather(g, b):
            pltpu.async_copy(
                table_hbm.at[idx_t.at[g]], rows.at[b], gsems[b]
            )

        def wait_gather(b):
            pltpu.make_async_copy(
                table_hbm.at[idx_t.at[0]], rows.at[b], gsems[b]
            ).wait()

        def repack(b):
            def body_r(r, c):
                for half in range(2):
                    for k in range(EMBED_DIM // VLEN):
                        packed[b, r, pl.ds(half * EMBED_DIM + k * VLEN, VLEN)] = (
                            rows[b, 2 * r + half, pl.ds(k * VLEN, VLEN)]
                        )
                return c

            lax.fori_loop(0, HGROUP, body_r, 0)

        def start_scatter(g, b):
            pltpu.async_copy(
                packed.at[b],
                out_hbm.at[pl.ds((gbase + g) * HGROUP, HGROUP)],
                ssems[b],
            )

        def wait_scatter(b):
            pltpu.make_async_copy(
                packed.at[b],
                out_hbm.at[pl.ds(gbase * HGROUP, HGROUP)],
                ssems[b],
            ).wait()

        for b in range(LEAD):
            start_gather(b, b)

        def loop(j, carry):
            for b in range(NBUF):
                wait_gather(b)

                @pl.when(j > 0)
                def _():
                    wait_scatter(b)

                repack(b)
                start_scatter(j * NBUF + b, b)
                b2 = (b + LEAD) % NBUF
                if b2 >= LEAD:
                    start_gather(j * NBUF + b2, b2)
                else:
                    @pl.when(j < NITER - 1)
                    def _():
                        start_gather((j + 1) * NBUF + b2, b2)

            return carry

        lax.fori_loop(0, NITER, loop, 0)

        for b in range(NBUF):
            wait_scatter(b)

    return body(table_pad, idx_groups)


@jax.jit
def kernel(input, W_embed, W_new):
    table = jnp.concatenate([W_embed, W_new], axis=0)
    table_pad = jnp.pad(table, ((0, 0), (0, PAD_DIM - EMBED_DIM)))
    idx_groups = input.astype(jnp.int32).reshape(BATCH * HIST // GROUP, GROUP)
    out = _sc_gather(table_pad, idx_groups)
    return out.reshape(BATCH, HIST, EMBED_DIM)
